# Initial kernel scaffold; baseline (speedup 1.0000x reference)
import jax
import jax.numpy as jnp
from jax.experimental import pallas as pl


def kernel(x, edge_index, batch, W1, b1, W2, b2, W3, b3):
    raise NotImplementedError("write your pallas kernel here")


# speedup vs baseline: 14.7582x; 14.7582x over previous
import functools

import jax
import jax.numpy as jnp
from jax import lax
from jax.experimental import pallas as pl
from jax.experimental.pallas import tpu as pltpu
from jax.experimental.pallas import tpu_sc as plsc

N = 10000
E = 320000
D = 128
H = 64
ATT = 16
G = 128
K = 3
ALPHA = 0.1

NC = 2
NS = 16
NW = NC * NS
CH = 128
NCHT = 79
PAD_E = NW * NCHT * CH
NR = 10240
RPT = NR // NS
NB_INIT = RPT // CH
DUMP0 = N + 48

@functools.cache
def _mesh():
    return plsc.VectorSubcoreMesh(
        core_axis_name="c", subcore_axis_name="s",
        num_cores=NC, num_subcores=NS)


def _deg_body(dstp, degp, idx_v, ones_v, zbuf, acc):
    c = lax.axis_index("c")
    s = lax.axis_index("s")
    wid = s * NC + c
    for i in range(CH // 16):
        ones_v[pl.ds(i * 16, 16)] = jnp.ones((16,), jnp.float32)
    for i in range(RPT // 16):
        zbuf[pl.ds(i * 16, 16)] = jnp.zeros((16,), jnp.float32)
    pltpu.sync_copy(zbuf, acc.at[pl.ds(s * RPT, RPT)])
    plsc.subcore_barrier()

    def body(j, _):
        pltpu.sync_copy(dstp.at[wid * NCHT + j], idx_v)
        pltpu.sync_copy(ones_v, acc.at[idx_v], add=True)
        return _

    lax.fori_loop(0, NCHT, body, None)
    plsc.subcore_barrier()
    pltpu.sync_copy(acc.at[pl.ds(s * RPT, RPT)], zbuf)
    pltpu.sync_copy(zbuf, degp.at[c, pl.ds(s * RPT, RPT)])


@functools.cache
def _deg_sc():
    return pl.kernel(
        _deg_body,
        out_type=jax.ShapeDtypeStruct((NC, NR), jnp.float32),
        mesh=_mesh(),
        scratch_types=[
            pltpu.VMEM((CH,), jnp.int32),
            pltpu.VMEM((CH,), jnp.float32),
            pltpu.VMEM((RPT,), jnp.float32),
            pltpu.VMEM_SHARED((NR,), jnp.float32),
        ],
    )


def _scat_body(g, srcp, dstp, zrows, p_out, sidx, didx, rows, stage, acc, sem):
    c = lax.axis_index("c")
    s = lax.axis_index("s")
    wid = s * NC + c

    @pl.when(c == 0)
    def _():
        for b in range(NB_INIT):
            r0 = s * RPT + b * CH
            pltpu.sync_copy(g.at[pl.ds(r0, CH)], stage)
            pltpu.sync_copy(stage, acc.at[pl.ds(r0, CH)])

    @pl.when(c != 0)
    def _():
        pltpu.sync_copy(zrows, stage)
        for b in range(NB_INIT):
            r0 = s * RPT + b * CH
            pltpu.sync_copy(stage, acc.at[pl.ds(r0, CH)])

    plsc.subcore_barrier()

    def body(j, _):
        cid = wid * NCHT + j
        pltpu.sync_copy(srcp.at[cid], sidx)
        pltpu.sync_copy(dstp.at[cid], didx)
        pltpu.async_copy(g.at[sidx], rows, sem).wait()
        pltpu.sync_copy(rows, acc.at[didx], add=True)
        return _

    lax.fori_loop(0, NCHT, body, None)
    plsc.subcore_barrier()

    for b in range(NB_INIT):
        r0 = s * RPT + b * CH
        pltpu.sync_copy(acc.at[pl.ds(r0, CH)], stage)
        pltpu.sync_copy(stage, p_out.at[c, pl.ds(r0, CH)])


@functools.cache
def _scat_sc():
    return pl.kernel(
        _scat_body,
        out_type=jax.ShapeDtypeStruct((NC, NR, H), jnp.float32),
        mesh=_mesh(),
        scratch_types=[
            pltpu.VMEM((CH,), jnp.int32),
            pltpu.VMEM((CH,), jnp.int32),
            pltpu.VMEM((CH, H), jnp.float32),
            pltpu.VMEM((CH, H), jnp.float32),
            pltpu.VMEM_SHARED((NR, H), jnp.float32),
            pltpu.SemaphoreType.DMA,
        ],
        compiler_params=pltpu.CompilerParams(use_tc_tiling_on_sc=False),
    )


_BR = 512
_NBLK = NR // _BR


def _prep_body(x_ref, w1_ref, degp_ref, g0_ref, d2_ref, ds_ref):
    pid = pl.program_id(0)
    deg = degp_ref[0] + degp_ref[1] + 1.0
    row = pid * _BR + lax.broadcasted_iota(jnp.int32, (_BR, 1), 0)
    real = (row < N).astype(jnp.float32)
    deg = deg.reshape(_BR, 1)
    d2_ref[...] = real / deg
    di = lax.rsqrt(deg)
    ds_ref[...] = real * deg * di
    y = jnp.dot(x_ref[...], w1_ref[...], preferred_element_type=jnp.float32)
    g0_ref[...] = real * di * y


def _prep_tc(xp, w1, degp):
    return pl.pallas_call(
        _prep_body,
        grid=(_NBLK,),
        in_specs=[
            pl.BlockSpec((_BR, D), lambda i: (i, 0)),
            pl.BlockSpec((D, H), lambda i: (0, 0)),
            pl.BlockSpec((NC, _BR), lambda i: (0, i)),
        ],
        out_specs=[
            pl.BlockSpec((_BR, H), lambda i: (i, 0)),
            pl.BlockSpec((_BR, 1), lambda i: (i, 0)),
            pl.BlockSpec((_BR, 1), lambda i: (i, 0)),
        ],
        out_shape=[
            jax.ShapeDtypeStruct((NR, H), jnp.float32),
            jax.ShapeDtypeStruct((NR, 1), jnp.float32),
            jax.ShapeDtypeStruct((NR, 1), jnp.float32),
        ],
    )(xp, w1, degp)


def _comb_body(p_ref, g0_ref, d2_ref, out_ref):
    s = p_ref[0] + p_ref[1]
    out_ref[...] = (1.0 - ALPHA) * d2_ref[...] * s + ALPHA * g0_ref[...]


def _comb_tc(p, g0, d2):
    return pl.pallas_call(
        _comb_body,
        grid=(_NBLK,),
        in_specs=[
            pl.BlockSpec((NC, _BR, H), lambda i: (0, i, 0)),
            pl.BlockSpec((_BR, H), lambda i: (i, 0)),
            pl.BlockSpec((_BR, 1), lambda i: (i, 0)),
        ],
        out_specs=pl.BlockSpec((_BR, H), lambda i: (i, 0)),
        out_shape=jax.ShapeDtypeStruct((NR, H), jnp.float32),
    )(p, g0, d2)


def _final_body(g3_ref, ds_ref, batch_ref, b1_ref, w2_ref, b2_ref, w3_ref,
                b3_ref, out_ref, accs, accc):
    pid = pl.program_id(0)

    @pl.when(pid == 0)
    def _():
        accs[...] = jnp.zeros_like(accs)
        accc[...] = jnp.zeros_like(accc)

    z = jnp.maximum(ds_ref[...] * g3_ref[...] + b1_ref[...], 0.0)
    onehot = (batch_ref[...] ==
              lax.broadcasted_iota(jnp.int32, (_BR, G), 1)).astype(jnp.float32)
    accs[...] += lax.dot_general(onehot, z, (((0,), (0,)), ((), ())),
                                 preferred_element_type=jnp.float32)
    accc[...] += lax.dot_general(onehot, jnp.ones((_BR, 1), jnp.float32),
                                 (((0,), (0,)), ((), ())),
                                 preferred_element_type=jnp.float32)

    @pl.when(pid == _NBLK - 1)
    def _():
        pooled = accs[...] / jnp.maximum(accc[...], 1.0)
        a = jnp.maximum(
            jnp.dot(pooled, w2_ref[...], preferred_element_type=jnp.float32)
            + b2_ref[...], 0.0)
        out_ref[...] = (
            jnp.dot(a, w3_ref[...], preferred_element_type=jnp.float32)
            + b3_ref[...])


def _final_tc(g3, dsq, batchp, b1, w2, b2, w3, b3):
    return pl.pallas_call(
        _final_body,
        grid=(_NBLK,),
        in_specs=[
            pl.BlockSpec((_BR, H), lambda i: (i, 0)),
            pl.BlockSpec((_BR, 1), lambda i: (i, 0)),
            pl.BlockSpec((_BR, 1), lambda i: (i, 0)),
            pl.BlockSpec((1, H), lambda i: (0, 0)),
            pl.BlockSpec((H, ATT), lambda i: (0, 0)),
            pl.BlockSpec((1, ATT), lambda i: (0, 0)),
            pl.BlockSpec((ATT, 1), lambda i: (0, 0)),
            pl.BlockSpec((1, 1), lambda i: (0, 0)),
        ],
        out_specs=pl.BlockSpec((G, 1), lambda i: (0, 0)),
        out_shape=jax.ShapeDtypeStruct((G, 1), jnp.float32),
        scratch_shapes=[
            pltpu.VMEM((G, H), jnp.float32),
            pltpu.VMEM((G, 1), jnp.float32),
        ],
    )(g3, dsq, batchp, b1, w2, b2, w3, b3)


def kernel(x, edge_index, batch, W1, b1, W2, b2, W3, b3):
    pad_ids = DUMP0 + (jnp.arange(PAD_E - E, dtype=jnp.int32) % CH)
    srcp = jnp.concatenate([edge_index[0], pad_ids]).reshape(NW * NCHT, CH)
    dstp = jnp.concatenate([edge_index[1], pad_ids]).reshape(NW * NCHT, CH)
    xp = jnp.pad(x, ((0, NR - N), (0, 0)))
    batchp = jnp.pad(batch, (0, NR - N), constant_values=G).reshape(NR, 1)
    zrows = jnp.zeros((CH, H), jnp.float32)

    degp = _deg_sc()(dstp)
    g0, d2, dsq = _prep_tc(xp, W1, degp)

    g = g0
    for _ in range(K):
        p = _scat_sc()(g, srcp, dstp, zrows)
        g = _comb_tc(p, g0, d2)

    out = _final_tc(g, dsq, batchp, b1.reshape(1, H), W2,
                    b2.reshape(1, ATT), W3, b3.reshape(1, 1))
    return out


# speedup vs baseline: 33.0927x; 2.2423x over previous
import functools

import jax
import jax.numpy as jnp
from jax import lax
from jax.experimental import pallas as pl
from jax.experimental.pallas import tpu as pltpu
from jax.experimental.pallas import tpu_sc as plsc

N = 10000
E = 320000
D = 128
H = 64
ATT = 16
G = 128
K = 3
ALPHA = 0.1

NC = 2
NS = 16
NW = NC * NS
CH = 128
NBUF = 8
NGRP = 10
NCHT = NBUF * NGRP
PAD_E = NW * NCHT * CH
NR = 10240
RPT = NR // NS
NB_INIT = RPT // CH
DUMP0 = N + 48

@functools.cache
def _mesh():
    return plsc.VectorSubcoreMesh(
        core_axis_name="c", subcore_axis_name="s",
        num_cores=NC, num_subcores=NS)


def _deg_body(dstp, degp, didx, ones_v, zbuf, acc, sem):
    c = lax.axis_index("c")
    s = lax.axis_index("s")
    wid = s * NC + c
    for i in range(CH // 16):
        ones_v[pl.ds(i * 16, 16)] = jnp.ones((16,), jnp.float32)
    for i in range(RPT // 16):
        zbuf[pl.ds(i * 16, 16)] = jnp.zeros((16,), jnp.float32)
    pltpu.async_copy(dstp.at[pl.ds(wid * NCHT, NCHT)], didx, sem).wait()
    pltpu.sync_copy(zbuf, acc.at[pl.ds(s * RPT, RPT)])
    plsc.subcore_barrier()

    def fire(j, _):
        pltpu.async_copy(ones_v, acc.at[didx.at[j]], sem, add=True)
        return _

    def drain(j, _):
        pltpu.make_async_copy(ones_v, acc.at[didx.at[0]], sem).wait()
        return _

    lax.fori_loop(0, NCHT, fire, None)
    lax.fori_loop(0, NCHT, drain, None)
    plsc.subcore_barrier()
    pltpu.sync_copy(acc.at[pl.ds(s * RPT, RPT)], zbuf)
    pltpu.sync_copy(zbuf, degp.at[c, pl.ds(s * RPT, RPT)])


@functools.cache
def _deg_sc():
    return pl.kernel(
        _deg_body,
        out_type=jax.ShapeDtypeStruct((NC, NR), jnp.float32),
        mesh=_mesh(),
        scratch_types=[
            pltpu.VMEM((NCHT, CH), jnp.int32),
            pltpu.VMEM((CH,), jnp.float32),
            pltpu.VMEM((RPT,), jnp.float32),
            pltpu.VMEM_SHARED((NR,), jnp.float32),
            pltpu.SemaphoreType.DMA,
        ],
    )


def _scat_body(g, srcp, dstp, zrows, p_out, sidx, didx, rows, acc, gsem, ssem):
    c = lax.axis_index("c")
    s = lax.axis_index("s")
    wid = s * NC + c

    pltpu.async_copy(srcp.at[pl.ds(wid * NCHT, NCHT)], sidx, gsem[0])
    pltpu.async_copy(dstp.at[pl.ds(wid * NCHT, NCHT)], didx, gsem[1])

    @pl.when(c == 0)
    def _():
        for b in range(NB_INIT):
            r0 = s * RPT + b * CH
            pltpu.sync_copy(g.at[pl.ds(r0, CH)], rows.at[b])
            pltpu.sync_copy(rows.at[b], acc.at[pl.ds(r0, CH)])

    @pl.when(c != 0)
    def _():
        pltpu.sync_copy(zrows, rows.at[0])
        for b in range(NB_INIT):
            r0 = s * RPT + b * CH
            pltpu.sync_copy(rows.at[0], acc.at[pl.ds(r0, CH)])

    pltpu.make_async_copy(srcp.at[pl.ds(0, NCHT)], sidx, gsem[0]).wait()
    pltpu.make_async_copy(dstp.at[pl.ds(0, NCHT)], didx, gsem[1]).wait()
    plsc.subcore_barrier()

    base = wid * NCHT

    for b in range(NBUF):
        pltpu.async_copy(g.at[sidx.at[b]], rows.at[b], gsem[b])

    def body(t, _):
        for b in range(NBUF):
            j = t * NBUF + b
            pltpu.make_async_copy(g.at[sidx.at[0]], rows.at[b],
                                  gsem[b]).wait()
            pltpu.async_copy(rows.at[b], acc.at[didx.at[j]], ssem[b],
                             add=True)
        for b in range(NBUF):
            pltpu.make_async_copy(rows.at[b], acc.at[didx.at[0]],
                                  ssem[b]).wait()

            @pl.when(t < NGRP - 1)
            def _():
                pltpu.async_copy(g.at[sidx.at[t * NBUF + b + NBUF]],
                                 rows.at[b], gsem[b])

        return _

    lax.fori_loop(0, NGRP, body, None)
    plsc.subcore_barrier()

    for b in range(NB_INIT):
        r0 = s * RPT + b * CH
        pltpu.sync_copy(acc.at[pl.ds(r0, CH)], rows.at[b])
        pltpu.sync_copy(rows.at[b], p_out.at[c, pl.ds(r0, CH)])


@functools.cache
def _scat_sc():
    return pl.kernel(
        _scat_body,
        out_type=jax.ShapeDtypeStruct((NC, NR, H), jnp.float32),
        mesh=_mesh(),
        scratch_types=[
            pltpu.VMEM((NCHT, CH), jnp.int32),
            pltpu.VMEM((NCHT, CH), jnp.int32),
            pltpu.VMEM((NBUF, CH, H), jnp.float32),
            pltpu.VMEM_SHARED((NR, H), jnp.float32),
            [pltpu.SemaphoreType.DMA] * NBUF,
            [pltpu.SemaphoreType.DMA] * NBUF,
        ],
        compiler_params=pltpu.CompilerParams(use_tc_tiling_on_sc=False),
    )


_BR = 512
_NBLK = NR // _BR


def _prep_body(x_ref, w1_ref, degp_ref, g0_ref, d2_ref, ds_ref):
    pid = pl.program_id(0)
    deg = degp_ref[0] + degp_ref[1] + 1.0
    row = pid * _BR + lax.broadcasted_iota(jnp.int32, (_BR, 1), 0)
    real = (row < N).astype(jnp.float32)
    deg = deg.reshape(_BR, 1)
    d2_ref[...] = real / deg
    di = lax.rsqrt(deg)
    ds_ref[...] = real * deg * di
    y = jnp.dot(x_ref[...], w1_ref[...], preferred_element_type=jnp.float32)
    g0_ref[...] = real * di * y


def _prep_tc(xp, w1, degp):
    return pl.pallas_call(
        _prep_body,
        grid=(_NBLK,),
        in_specs=[
            pl.BlockSpec((_BR, D), lambda i: (i, 0)),
            pl.BlockSpec((D, H), lambda i: (0, 0)),
            pl.BlockSpec((NC, _BR), lambda i: (0, i)),
        ],
        out_specs=[
            pl.BlockSpec((_BR, H), lambda i: (i, 0)),
            pl.BlockSpec((_BR, 1), lambda i: (i, 0)),
            pl.BlockSpec((_BR, 1), lambda i: (i, 0)),
        ],
        out_shape=[
            jax.ShapeDtypeStruct((NR, H), jnp.float32),
            jax.ShapeDtypeStruct((NR, 1), jnp.float32),
            jax.ShapeDtypeStruct((NR, 1), jnp.float32),
        ],
    )(xp, w1, degp)


def _comb_body(p_ref, g0_ref, d2_ref, out_ref):
    s = p_ref[0] + p_ref[1]
    out_ref[...] = (1.0 - ALPHA) * d2_ref[...] * s + ALPHA * g0_ref[...]


def _comb_tc(p, g0, d2):
    return pl.pallas_call(
        _comb_body,
        grid=(_NBLK,),
        in_specs=[
            pl.BlockSpec((NC, _BR, H), lambda i: (0, i, 0)),
            pl.BlockSpec((_BR, H), lambda i: (i, 0)),
            pl.BlockSpec((_BR, 1), lambda i: (i, 0)),
        ],
        out_specs=pl.BlockSpec((_BR, H), lambda i: (i, 0)),
        out_shape=jax.ShapeDtypeStruct((NR, H), jnp.float32),
    )(p, g0, d2)


def _final_body(g3_ref, ds_ref, batch_ref, b1_ref, w2_ref, b2_ref, w3_ref,
                b3_ref, out_ref, accs, accc):
    pid = pl.program_id(0)

    @pl.when(pid == 0)
    def _():
        accs[...] = jnp.zeros_like(accs)
        accc[...] = jnp.zeros_like(accc)

    z = jnp.maximum(ds_ref[...] * g3_ref[...] + b1_ref[...], 0.0)
    onehot = (batch_ref[...] ==
              lax.broadcasted_iota(jnp.int32, (_BR, G), 1)).astype(jnp.float32)
    accs[...] += lax.dot_general(onehot, z, (((0,), (0,)), ((), ())),
                                 preferred_element_type=jnp.float32)
    accc[...] += lax.dot_general(onehot, jnp.ones((_BR, 1), jnp.float32),
                                 (((0,), (0,)), ((), ())),
                                 preferred_element_type=jnp.float32)

    @pl.when(pid == _NBLK - 1)
    def _():
        pooled = accs[...] / jnp.maximum(accc[...], 1.0)
        a = jnp.maximum(
            jnp.dot(pooled, w2_ref[...], preferred_element_type=jnp.float32)
            + b2_ref[...], 0.0)
        out_ref[...] = (
            jnp.dot(a, w3_ref[...], preferred_element_type=jnp.float32)
            + b3_ref[...])


def _final_tc(g3, dsq, batchp, b1, w2, b2, w3, b3):
    return pl.pallas_call(
        _final_body,
        grid=(_NBLK,),
        in_specs=[
            pl.BlockSpec((_BR, H), lambda i: (i, 0)),
            pl.BlockSpec((_BR, 1), lambda i: (i, 0)),
            pl.BlockSpec((_BR, 1), lambda i: (i, 0)),
            pl.BlockSpec((1, H), lambda i: (0, 0)),
            pl.BlockSpec((H, ATT), lambda i: (0, 0)),
            pl.BlockSpec((1, ATT), lambda i: (0, 0)),
            pl.BlockSpec((ATT, 1), lambda i: (0, 0)),
            pl.BlockSpec((1, 1), lambda i: (0, 0)),
        ],
        out_specs=pl.BlockSpec((G, 1), lambda i: (0, 0)),
        out_shape=jax.ShapeDtypeStruct((G, 1), jnp.float32),
        scratch_shapes=[
            pltpu.VMEM((G, H), jnp.float32),
            pltpu.VMEM((G, 1), jnp.float32),
        ],
    )(g3, dsq, batchp, b1, w2, b2, w3, b3)


def kernel(x, edge_index, batch, W1, b1, W2, b2, W3, b3):
    pad_ids = DUMP0 + (jnp.arange(PAD_E - E, dtype=jnp.int32) % CH)
    srcp = jnp.concatenate([edge_index[0], pad_ids]).reshape(NW * NCHT, CH)
    dstp = jnp.concatenate([edge_index[1], pad_ids]).reshape(NW * NCHT, CH)
    xp = jnp.pad(x, ((0, NR - N), (0, 0)))
    batchp = jnp.pad(batch, (0, NR - N), constant_values=G).reshape(NR, 1)
    zrows = jnp.zeros((CH, H), jnp.float32)

    degp = _deg_sc()(dstp)
    g0, d2, dsq = _prep_tc(xp, W1, degp)

    g = g0
    for _ in range(K):
        p = _scat_sc()(g, srcp, dstp, zrows)
        g = _comb_tc(p, g0, d2)

    out = _final_tc(g, dsq, batchp, b1.reshape(1, H), W2,
                    b2.reshape(1, ATT), W3, b3.reshape(1, 1))
    return out


# speedup vs baseline: 40.3213x; 1.2184x over previous
import functools

import jax
import jax.numpy as jnp
from jax import lax
from jax.experimental import pallas as pl
from jax.experimental.pallas import tpu as pltpu
from jax.experimental.pallas import tpu_sc as plsc

N = 10000
E = 320000
D = 128
H = 64
ATT = 16
G = 128
K = 3
ALPHA = 0.1

NC = 2
NS = 16
HC = H // NC
CH = 128
NBUF = 8
NGRP = 20
NCHT = NBUF * NGRP
PAD_E = NS * NCHT * CH
NR = 10240
RPT = NR // NS
NB_INIT = RPT // CH
DUMP0 = N + 48


@functools.cache
def _mesh():
    return plsc.VectorSubcoreMesh(
        core_axis_name="c", subcore_axis_name="s",
        num_cores=NC, num_subcores=NS)


def _deg_body(dstp, degp, didx, ones_v, zbuf, acc, sem):
    c = lax.axis_index("c")
    s = lax.axis_index("s")
    wid = s * NC + c
    ncht_half = NCHT // 2
    for i in range(CH // 16):
        ones_v[pl.ds(i * 16, 16)] = jnp.ones((16,), jnp.float32)
    for i in range(RPT // 16):
        zbuf[pl.ds(i * 16, 16)] = jnp.zeros((16,), jnp.float32)
    pltpu.async_copy(dstp.at[pl.ds(wid * ncht_half, ncht_half)], didx,
                     sem).wait()
    pltpu.sync_copy(zbuf, acc.at[pl.ds(s * RPT, RPT)])
    plsc.subcore_barrier()

    def fire(j, _):
        pltpu.async_copy(ones_v, acc.at[didx.at[j]], sem, add=True)
        return _

    def drain(j, _):
        pltpu.make_async_copy(ones_v, acc.at[didx.at[0]], sem).wait()
        return _

    lax.fori_loop(0, ncht_half, fire, None)
    lax.fori_loop(0, ncht_half, drain, None)
    plsc.subcore_barrier()
    pltpu.sync_copy(acc.at[pl.ds(s * RPT, RPT)], zbuf)
    pltpu.sync_copy(zbuf, degp.at[c, pl.ds(s * RPT, RPT)])


@functools.cache
def _deg_sc():
    return pl.kernel(
        _deg_body,
        out_type=jax.ShapeDtypeStruct((NC, NR), jnp.float32),
        mesh=_mesh(),
        scratch_types=[
            pltpu.VMEM((NCHT // 2, CH), jnp.int32),
            pltpu.VMEM((CH,), jnp.float32),
            pltpu.VMEM((RPT,), jnp.float32),
            pltpu.VMEM_SHARED((NR,), jnp.float32),
            pltpu.SemaphoreType.DMA,
        ],
    )


def _appnp_body(g0c, d2, srcp, dstp, g3c, gbuf, sidx, didx, rows, d2v, acc,
                isem, gsem, ssem):
    c = lax.axis_index("c")
    s = lax.axis_index("s")

    pltpu.async_copy(srcp.at[pl.ds(s * NCHT, NCHT)], sidx, isem)
    pltpu.async_copy(dstp.at[pl.ds(s * NCHT, NCHT)], didx, isem)
    pltpu.async_copy(d2.at[pl.ds(s * RPT, RPT)], d2v, isem)

    for b in range(NB_INIT):
        r0 = s * RPT + b * CH
        pltpu.sync_copy(g0c.at[c, pl.ds(r0, CH)], rows.at[b])
        pltpu.sync_copy(rows.at[b], acc.at[pl.ds(r0, CH)])

    pltpu.make_async_copy(srcp.at[pl.ds(0, NCHT)], sidx, isem).wait()
    pltpu.make_async_copy(dstp.at[pl.ds(0, NCHT)], didx, isem).wait()
    pltpu.make_async_copy(d2.at[pl.ds(0, RPT)], d2v, isem).wait()
    plsc.subcore_barrier()

    for r in range(K):
        gsrc = g0c if r == 0 else gbuf
        gdst = gbuf if r < K - 1 else g3c

        for b in range(NBUF):
            pltpu.async_copy(gsrc.at[c].at[sidx.at[b]], rows.at[b], gsem[b])

        def body(t, _, gsrc=gsrc):
            for b in range(NBUF):
                j = t * NBUF + b
                pltpu.make_async_copy(gsrc.at[c].at[sidx.at[0]], rows.at[b],
                                      gsem[b]).wait()
                pltpu.async_copy(rows.at[b], acc.at[didx.at[j]], ssem[b],
                                 add=True)
            for b in range(NBUF):
                pltpu.make_async_copy(rows.at[b], acc.at[didx.at[0]],
                                      ssem[b]).wait()

                @pl.when(t < NGRP - 1)
                def _():
                    pltpu.async_copy(gsrc.at[c].at[sidx.at[t * NBUF + b + NBUF]],
                                     rows.at[b], gsem[b])

            return _

        lax.fori_loop(0, NGRP, body, None)
        plsc.subcore_barrier()

        for b in range(NB_INIT):
            r0 = s * RPT + b * CH
            pltpu.sync_copy(acc.at[pl.ds(r0, CH)], rows.at[0])
            pltpu.sync_copy(g0c.at[c, pl.ds(r0, CH)], rows.at[1])

            def crow(i, _, b=b):
                dscale = plsc.load_gather(
                    d2v, [jnp.full((16,), b * CH + i, jnp.int32)])
                for half in range(HC // 16):
                    sl = pl.ds(half * 16, 16)
                    rows.at[2][i, sl] = (dscale * rows.at[0][i, sl]
                                         + ALPHA * rows.at[1][i, sl])
                return _

            lax.fori_loop(0, CH, crow, None)
            pltpu.sync_copy(rows.at[2], gdst.at[c, pl.ds(r0, CH)])
            if r < K - 1:
                pltpu.sync_copy(rows.at[2], acc.at[pl.ds(r0, CH)])
        plsc.subcore_barrier()


@functools.cache
def _appnp_sc():
    return pl.kernel(
        _appnp_body,
        out_type=(jax.ShapeDtypeStruct((NC, NR, HC), jnp.float32),
                  jax.ShapeDtypeStruct((NC, NR, HC), jnp.float32)),
        mesh=_mesh(),
        scratch_types=[
            pltpu.VMEM((NCHT, CH), jnp.int32),
            pltpu.VMEM((NCHT, CH), jnp.int32),
            pltpu.VMEM((NBUF, CH, HC), jnp.float32),
            pltpu.VMEM((RPT,), jnp.float32),
            pltpu.VMEM_SHARED((NR, HC), jnp.float32),
            pltpu.SemaphoreType.DMA,
            [pltpu.SemaphoreType.DMA] * NBUF,
            [pltpu.SemaphoreType.DMA] * NBUF,
        ],
        compiler_params=pltpu.CompilerParams(use_tc_tiling_on_sc=False,
                                             needs_layout_passes=False),
    )


_BR = 512
_NBLK = NR // _BR


def _prep_body(x_ref, w1_ref, degp_ref, g0_ref, d2_ref, ds_ref):
    pid = pl.program_id(0)
    deg = degp_ref[0] + degp_ref[1] + 1.0
    row = pid * _BR + lax.broadcasted_iota(jnp.int32, (_BR, 1), 0)
    real = (row < N).astype(jnp.float32)
    deg = deg.reshape(_BR, 1)
    d2_ref[...] = (1.0 - ALPHA) * real / deg
    di = lax.rsqrt(deg)
    ds_ref[...] = real * deg * di
    y = jnp.dot(x_ref[...], w1_ref[...], preferred_element_type=jnp.float32)
    g0 = real * di * y
    g0_ref[0] = g0[:, :HC]
    g0_ref[1] = g0[:, HC:]


def _prep_tc(xp, w1, degp):
    return pl.pallas_call(
        _prep_body,
        grid=(_NBLK,),
        in_specs=[
            pl.BlockSpec((_BR, D), lambda i: (i, 0)),
            pl.BlockSpec((D, H), lambda i: (0, 0)),
            pl.BlockSpec((NC, _BR), lambda i: (0, i)),
        ],
        out_specs=[
            pl.BlockSpec((NC, _BR, HC), lambda i: (0, i, 0)),
            pl.BlockSpec((_BR, 1), lambda i: (i, 0)),
            pl.BlockSpec((_BR, 1), lambda i: (i, 0)),
        ],
        out_shape=[
            jax.ShapeDtypeStruct((NC, NR, HC), jnp.float32),
            jax.ShapeDtypeStruct((NR, 1), jnp.float32),
            jax.ShapeDtypeStruct((NR, 1), jnp.float32),
        ],
    )(xp, w1, degp)


def _final_body(g3_ref, ds_ref, batch_ref, b1_ref, w2_ref, b2_ref, w3_ref,
                b3_ref, out_ref, accs, accc):
    pid = pl.program_id(0)

    @pl.when(pid == 0)
    def _():
        accs[...] = jnp.zeros_like(accs)
        accc[...] = jnp.zeros_like(accc)

    g3 = jnp.concatenate([g3_ref[0], g3_ref[1]], axis=1)
    z = jnp.maximum(ds_ref[...] * g3 + b1_ref[...], 0.0)
    onehot = (batch_ref[...] ==
              lax.broadcasted_iota(jnp.int32, (_BR, G), 1)).astype(jnp.float32)
    accs[...] += lax.dot_general(onehot, z, (((0,), (0,)), ((), ())),
                                 preferred_element_type=jnp.float32)
    accc[...] += lax.dot_general(onehot, jnp.ones((_BR, 1), jnp.float32),
                                 (((0,), (0,)), ((), ())),
                                 preferred_element_type=jnp.float32)

    @pl.when(pid == _NBLK - 1)
    def _():
        pooled = accs[...] / jnp.maximum(accc[...], 1.0)
        a = jnp.maximum(
            jnp.dot(pooled, w2_ref[...], preferred_element_type=jnp.float32)
            + b2_ref[...], 0.0)
        out_ref[...] = (
            jnp.dot(a, w3_ref[...], preferred_element_type=jnp.float32)
            + b3_ref[...])


def _final_tc(g3c, dsq, batchp, b1, w2, b2, w3, b3):
    return pl.pallas_call(
        _final_body,
        grid=(_NBLK,),
        in_specs=[
            pl.BlockSpec((NC, _BR, HC), lambda i: (0, i, 0)),
            pl.BlockSpec((_BR, 1), lambda i: (i, 0)),
            pl.BlockSpec((_BR, 1), lambda i: (i, 0)),
            pl.BlockSpec((1, H), lambda i: (0, 0)),
            pl.BlockSpec((H, ATT), lambda i: (0, 0)),
            pl.BlockSpec((1, ATT), lambda i: (0, 0)),
            pl.BlockSpec((ATT, 1), lambda i: (0, 0)),
            pl.BlockSpec((1, 1), lambda i: (0, 0)),
        ],
        out_specs=pl.BlockSpec((G, 1), lambda i: (0, 0)),
        out_shape=jax.ShapeDtypeStruct((G, 1), jnp.float32),
        scratch_shapes=[
            pltpu.VMEM((G, H), jnp.float32),
            pltpu.VMEM((G, 1), jnp.float32),
        ],
    )(g3c, dsq, batchp, b1, w2, b2, w3, b3)


def kernel(x, edge_index, batch, W1, b1, W2, b2, W3, b3):
    pad_ids = DUMP0 + (jnp.arange(PAD_E - E, dtype=jnp.int32) % CH)
    srcp = jnp.concatenate([edge_index[0], pad_ids]).reshape(NS * NCHT, CH)
    dstp = jnp.concatenate([edge_index[1], pad_ids]).reshape(NS * NCHT, CH)
    xp = jnp.pad(x, ((0, NR - N), (0, 0)))
    batchp = jnp.pad(batch, (0, NR - N), constant_values=G).reshape(NR, 1)

    degp = _deg_sc()(dstp)
    g0c, d2, dsq = _prep_tc(xp, W1, degp)
    g3c, _ = _appnp_sc()(g0c, d2.reshape(NR), srcp, dstp)
    out = _final_tc(g3c, dsq, batchp, b1.reshape(1, H), W2,
                    b2.reshape(1, ATT), W3, b3.reshape(1, 1))
    return out
